# baseline (speedup 1.0000x reference)
import jax
import jax.numpy as jnp
from jax import lax
from jax.experimental import pallas as pl
from jax.experimental.pallas import tpu as pltpu
import jax.experimental.pallas.tpu_sc as plsc

N = 10000
E = 320000
DIN = 128
DH = 16
HEADS = 8
G = 64

NC = 2
NS = 16
CH = 32
CHUNKS = 324
EPT = CH * CHUNKS
EPAD = EPT * NC * NS

NB = 10112
RPTB = NB // NS
ROWBLK = 128
NBLKS = NB // ROWBLK

DRB = 1280
DRPT = DRB // NS
NBT = NB + DRB

ND = 10240
RPTD = ND // NS

_mesh = plsc.VectorSubcoreMesh(core_axis_name="c", subcore_axis_name="s",
                               num_cores=NC, num_subcores=NS)
_sc_params = pltpu.CompilerParams(needs_layout_passes=False)


def _mm_body(x_ref, wl_ref, wr_ref, ol_ref, or_ref):
    xb = x_ref[...]
    ol_ref[...] = jnp.dot(xb, wl_ref[...], preferred_element_type=jnp.float32)
    or_ref[...] = jnp.dot(xb, wr_ref[...], preferred_element_type=jnp.float32)


def _stage_a(x_pad, W1l, W1r):
    return pl.pallas_call(
        _mm_body,
        grid=(NBLKS,),
        in_specs=[
            pl.BlockSpec((ROWBLK, DIN), lambda i: (i, 0)),
            pl.BlockSpec((DIN, HEADS * DH), lambda i: (0, 0)),
            pl.BlockSpec((DIN, HEADS * DH), lambda i: (0, 0)),
        ],
        out_specs=[
            pl.BlockSpec((ROWBLK, HEADS * DH), lambda i: (i, 0)),
            pl.BlockSpec((ROWBLK, HEADS * DH), lambda i: (i, 0)),
        ],
        out_shape=[
            jax.ShapeDtypeStruct((NB, HEADS * DH), jnp.float32),
            jax.ShapeDtypeStruct((NB, HEADS * DH), jnp.float32),
        ],
    )(x_pad, W1l, W1r)


def _sc1_body(xl_hbm, xr_hbm, src_hbm, dst_hbm, att_hbm,
              accp_out,
              srcb, dstb, dstb2, xlrows, xrrows, cbuf2, exbuf, attv, semx, semr,
              acc_sh):
    c = lax.axis_index("c")
    s = lax.axis_index("s")

    pltpu.sync_copy(att_hbm, attv)

    def zrow(i, carry):
        for q in range(8):
            cbuf2[i, pl.ds(16 * q, 16)] = jnp.zeros((16,), jnp.float32)
        return carry

    lax.fori_loop(0, CH, zrow, 0)
    for t in range(RPTB // CH):
        pltpu.sync_copy(cbuf2, acc_sh.at[pl.ds(s * RPTB + t * CH, CH)])
    rem = RPTB % CH
    if rem:
        off = RPTB - rem
        pltpu.sync_copy(cbuf2.at[pl.ds(0, rem)],
                        acc_sh.at[pl.ds(s * RPTB + off, rem)])
    dbase = NB + s * DRPT
    for t in range(DRPT // CH):
        pltpu.sync_copy(cbuf2, acc_sh.at[pl.ds(dbase + t * CH, CH)])
    drem = DRPT % CH
    if drem:
        pltpu.sync_copy(cbuf2.at[pl.ds(0, drem)],
                        acc_sh.at[pl.ds(dbase + DRPT - drem, drem)])
    plsc.subcore_barrier()

    attrows = [attv[pl.ds(h * DH, DH)] for h in range(HEADS)]
    lane = lax.iota(jnp.int32, 16)

    base0 = c * (NS * EPT) + s * EPT
    zv16 = jnp.zeros((16,), jnp.float32)

    def fetch(k, p):
        base = base0 + k * CH
        pltpu.sync_copy(src_hbm.at[pl.ds(base, CH)], srcb[p])
        pltpu.sync_copy(dst_hbm.at[pl.ds(base, CH)], dstb[p])
        pltpu.async_copy(xl_hbm.at[srcb[p]], xlrows[p], semx[p])
        pltpu.async_copy(xr_hbm.at[dstb[p]], xrrows[p], semr[p])

    def process(p):
        pltpu.make_async_copy(xl_hbm.at[srcb[p]], xlrows[p], semx[p]).wait()
        pltpu.make_async_copy(xr_hbm.at[dstb[p]], xrrows[p], semr[p]).wait()
        xlr = xlrows[p]
        xrr = xrrows[p]
        dsb = dstb[p]

        def group(g, carry2):
            dstv = dsb[pl.ds(g * 16, 16)]
            dstb2[pl.ds(g * 16, 16)] = NB + lax.shift_right_logical(dstv, 3)
            for l in range(16):
                j = g * 16 + l
                sels = []
                for h in range(HEADS):
                    sl = pl.ds(16 * h, 16)
                    m = xlr[j, sl] + xrr[j, sl]
                    lk = jnp.maximum(m, 0.2 * m)
                    eh = jnp.sum(lk * attrows[h])
                    sels.append(jnp.where(lane == h, eh, 0.0))
                acc = (((sels[0] + sels[1]) + (sels[2] + sels[3]))
                       + ((sels[4] + sels[5]) + (sels[6] + sels[7])))
                exbuf[j, pl.ds(0, 16)] = jnp.exp(acc)
            for l in range(16):
                j = g * 16 + l
                bj = lax.rem(dstv[l], 8)
                exv = exbuf[j, pl.ds(0, 16)]
                for h in range(HEADS):
                    sl = pl.ds(16 * h, 16)
                    xlr[j, sl] = xlr[j, sl] * exv[h]
                for b in range(8):
                    cbuf2[j, pl.ds(16 * b, 16)] = jnp.where(bj == b, exv, zv16)
            return carry2

        lax.fori_loop(0, CH // 16, group, 0)

        pltpu.sync_copy(xlr, acc_sh.at[dsb], add=True)
        pltpu.sync_copy(cbuf2, acc_sh.at[dstb2], add=True)

    fetch(0, 0)

    def chunk2(k2, carry):
        fetch(2 * k2 + 1, 1)
        process(0)

        @pl.when(k2 + 1 < CHUNKS // 2)
        def _():
            fetch(2 * k2 + 2, 0)

        process(1)
        return carry

    lax.fori_loop(0, CHUNKS // 2, chunk2, 0)
    plsc.subcore_barrier()

    row0 = c * NBT + s * RPTB
    pltpu.sync_copy(acc_sh.at[pl.ds(s * RPTB, RPTB)], accp_out.at[pl.ds(row0, RPTB)])
    drow0 = c * NBT + NB + s * DRPT
    pltpu.sync_copy(acc_sh.at[pl.ds(NB + s * DRPT, DRPT)],
                    accp_out.at[pl.ds(drow0, DRPT)])


def _stage_b(xl, xr, src_pad, dst_pad, att1):
    f = pl.kernel(
        _sc1_body,
        out_type=jax.ShapeDtypeStruct((NC * NBT, HEADS * DH), jnp.float32),
        mesh=_mesh,
        compiler_params=_sc_params,
        scratch_types=[
            [pltpu.VMEM((CH,), jnp.int32)] * 2,
            [pltpu.VMEM((CH,), jnp.int32)] * 2,
            pltpu.VMEM((CH,), jnp.int32),
            [pltpu.VMEM((CH, HEADS * DH), jnp.float32)] * 2,
            [pltpu.VMEM((CH, HEADS * DH), jnp.float32)] * 2,
            pltpu.VMEM((CH, HEADS * DH), jnp.float32),
            pltpu.VMEM((CH, 16), jnp.float32),
            pltpu.VMEM((HEADS * DH,), jnp.float32),
            [pltpu.SemaphoreType.DMA] * 2,
            [pltpu.SemaphoreType.DMA] * 2,
            pltpu.VMEM_SHARED((NBT, HEADS * DH), jnp.float32),
        ],
    )
    return f(xl, xr, src_pad, dst_pad, att1)


def _c_body(np_ref, dp_ref, b1_ref, w2_ref, hcat_ref):
    num = np_ref[0] + np_ref[1]
    den = (dp_ref[0] + dp_ref[1])[:, :HEADS]
    den16 = jnp.reshape(
        jnp.broadcast_to(den[:, :, None], (ROWBLK, HEADS, DH)),
        (ROWBLK, HEADS * DH))
    h = num / (den16 + 1e-16) + b1_ref[...]
    h = jnp.where(h > 0, h, jnp.exp(jnp.minimum(h, 0.0)) - 1.0)
    hcat_ref[...] = jnp.dot(h, w2_ref[...], preferred_element_type=jnp.float32)


def _stage_c(nump, denp, b1, W2cat):
    return pl.pallas_call(
        _c_body,
        grid=(NBLKS,),
        in_specs=[
            pl.BlockSpec((NC, ROWBLK, HEADS * DH), lambda i: (0, i, 0)),
            pl.BlockSpec((NC, ROWBLK, 16), lambda i: (0, i, 0)),
            pl.BlockSpec((1, HEADS * DH), lambda i: (0, 0)),
            pl.BlockSpec((HEADS * DH, 2), lambda i: (0, 0)),
        ],
        out_specs=pl.BlockSpec((ROWBLK, 2), lambda i: (i, 0)),
        out_shape=jax.ShapeDtypeStruct((NB, 2), jnp.float32),
    )(nump, denp, b1, W2cat)


def _sc2_body(hl_hbm, hr_hbm, src_hbm, dst_hbm, att2_hbm,
              num2_out, den2_out,
              hlv, hrv, srcb, dstb, exb2, cnb, attb, zv, sem1,
              num2_sh, den2_sh):
    c = lax.axis_index("c")
    s = lax.axis_index("s")

    pltpu.sync_copy(hl_hbm, hlv)
    pltpu.sync_copy(hr_hbm, hrv)
    pltpu.sync_copy(att2_hbm, attb)

    def zinit(i, carry):
        zv[pl.ds(i * 16, 16)] = jnp.zeros((16,), jnp.float32)
        return carry

    lax.fori_loop(0, RPTD // 16, zinit, 0)
    pltpu.sync_copy(zv, num2_sh.at[pl.ds(s * RPTD, RPTD)])
    pltpu.sync_copy(zv, den2_sh.at[pl.ds(s * RPTD, RPTD)])
    plsc.subcore_barrier()

    att2v = attb[...]
    base0 = c * (NS * EPT) + s * EPT

    def chunk(k, carry):
        base = base0 + k * CH
        pltpu.sync_copy(src_hbm.at[pl.ds(base, CH)], srcb)
        pltpu.sync_copy(dst_hbm.at[pl.ds(base, CH)], dstb)

        def group(g, carry2):
            sl = pl.ds(g * 16, 16)
            sv = srcb[sl]
            dv = dstb[sl]
            a = plsc.load_gather(hlv, [sv])
            b = plsc.load_gather(hrv, [dv])
            m = a + b
            lk = jnp.maximum(m, 0.2 * m)
            ex = jnp.exp(lk * att2v)
            exb2[sl] = ex
            cnb[sl] = ex * a
            return carry2

        lax.fori_loop(0, CH // 16, group, 0)

        pltpu.sync_copy(cnb, num2_sh.at[dstb], add=True)
        pltpu.sync_copy(exb2, den2_sh.at[dstb], add=True)
        return carry

    lax.fori_loop(0, CHUNKS, chunk, 0)
    plsc.subcore_barrier()

    row0 = c * ND + s * RPTD
    pltpu.sync_copy(num2_sh.at[pl.ds(s * RPTD, RPTD)], num2_out.at[pl.ds(row0, RPTD)])
    pltpu.sync_copy(den2_sh.at[pl.ds(s * RPTD, RPTD)], den2_out.at[pl.ds(row0, RPTD)])


def _stage_d(hl, hr, src_pad, dst_pad, att2v):
    f = pl.kernel(
        _sc2_body,
        out_type=[
            jax.ShapeDtypeStruct((NC * ND,), jnp.float32),
            jax.ShapeDtypeStruct((NC * ND,), jnp.float32),
        ],
        mesh=_mesh,
        compiler_params=_sc_params,
        scratch_types=[
            pltpu.VMEM((ND,), jnp.float32),
            pltpu.VMEM((ND,), jnp.float32),
            pltpu.VMEM((CH,), jnp.int32),
            pltpu.VMEM((CH,), jnp.int32),
            pltpu.VMEM((CH,), jnp.float32),
            pltpu.VMEM((CH,), jnp.float32),
            pltpu.VMEM((16,), jnp.float32),
            pltpu.VMEM((RPTD,), jnp.float32),
            pltpu.SemaphoreType.DMA,
            pltpu.VMEM_SHARED((ND,), jnp.float32),
            pltpu.VMEM_SHARED((ND,), jnp.float32),
        ],
    )
    return f(hl, hr, src_pad, dst_pad, att2v)


def _e_body(n2_ref, d2_ref, b2_ref, batch_ref, o_ref):
    nv = jnp.sum(n2_ref[...], axis=0, keepdims=True)
    dv = jnp.sum(d2_ref[...], axis=0, keepdims=True)
    nodeval = nv / (dv + 1e-16) + b2_ref[...]
    gids = lax.broadcasted_iota(jnp.int32, (G, ND), 0)
    oh = (gids == batch_ref[...]).astype(jnp.float32)
    out = lax.dot_general(oh, nodeval, (((1,), (1,)), ((), ())),
                          preferred_element_type=jnp.float32)
    o_ref[...] = out


def _stage_e(num2p, den2p, b2, batch_pad):
    return pl.pallas_call(
        _e_body,
        out_shape=jax.ShapeDtypeStruct((G, 1), jnp.float32),
    )(num2p, den2p, b2, batch_pad)


def kernel(x, edge_index, batch, edge_attr, W1l, W1r, att1, b1, W2l, W2r, att2, b2):
    loop = jnp.arange(N, dtype=edge_index.dtype)
    padv = jnp.full((EPAD - (E + N),), N, dtype=edge_index.dtype)
    src_pad = jnp.concatenate([edge_index[0], loop, padv])
    dst_pad = jnp.concatenate([edge_index[1], loop, padv])

    x_pad = jnp.pad(x, ((0, NB - N), (0, 0)))
    att2v = jnp.full((16,), att2[0, 0], jnp.float32)
    W2cat = jnp.concatenate([W2l, W2r], axis=1)
    batch_pad = jnp.concatenate(
        [batch, jnp.full((ND - N,), G, batch.dtype)]).reshape(1, ND)

    xl, xr = _stage_a(x_pad, W1l, W1r)

    accp = _stage_b(xl, xr, src_pad, dst_pad, att1.reshape(HEADS * DH))
    accp = accp.reshape(NC, NBT, HEADS * DH)
    nump = accp[:, :NB, :]
    denp = accp[:, NB:, :].reshape(NC, DRB * 8, 16)[:, :NB, :]

    hcat = _stage_c(nump, denp, b1.reshape(1, HEADS * DH), W2cat)
    hl = jnp.pad(hcat[:, 0], (0, ND - NB))
    hr = jnp.pad(hcat[:, 1], (0, ND - NB))

    num2p, den2p = _stage_d(hl, hr, src_pad, dst_pad, att2v)

    out = _stage_e(num2p.reshape(NC, ND), den2p.reshape(NC, ND),
                   b2.reshape(1, 1), batch_pad)
    return out


# speedup vs baseline: 1.1590x; 1.1590x over previous
import jax
import jax.numpy as jnp
from jax import lax
from jax.experimental import pallas as pl
from jax.experimental.pallas import tpu as pltpu
import jax.experimental.pallas.tpu_sc as plsc

N = 10000
E = 320000
DIN = 128
DH = 16
HEADS = 8
G = 64

NC = 2
NS = 16
CH = 64
CHUNKS = 162
CHD = 128
CHUNKSD = 81
EPT = CH * CHUNKS
EPAD = EPT * NC * NS

NB = 10112
RPTB = NB // NS
ROWBLK = 128
NBLKS = NB // ROWBLK

DRB = 1280
DRPT = DRB // NS
NBT = NB + DRB

ND = 10240
RPTD = ND // NS

_mesh = plsc.VectorSubcoreMesh(core_axis_name="c", subcore_axis_name="s",
                               num_cores=NC, num_subcores=NS)
_sc_params = pltpu.CompilerParams(needs_layout_passes=False)


def _mm_body(x_ref, wl_ref, wr_ref, ol_ref, or_ref):
    xb = x_ref[...]
    ol_ref[...] = jnp.dot(xb, wl_ref[...], preferred_element_type=jnp.float32)
    or_ref[...] = jnp.dot(xb, wr_ref[...], preferred_element_type=jnp.float32)


def _stage_a(x_pad, W1l, W1r):
    return pl.pallas_call(
        _mm_body,
        grid=(NBLKS,),
        in_specs=[
            pl.BlockSpec((ROWBLK, DIN), lambda i: (i, 0)),
            pl.BlockSpec((DIN, HEADS * DH), lambda i: (0, 0)),
            pl.BlockSpec((DIN, HEADS * DH), lambda i: (0, 0)),
        ],
        out_specs=[
            pl.BlockSpec((ROWBLK, HEADS * DH), lambda i: (i, 0)),
            pl.BlockSpec((ROWBLK, HEADS * DH), lambda i: (i, 0)),
        ],
        out_shape=[
            jax.ShapeDtypeStruct((NB, HEADS * DH), jnp.float32),
            jax.ShapeDtypeStruct((NB, HEADS * DH), jnp.float32),
        ],
    )(x_pad, W1l, W1r)


def _sc1_body(xl_hbm, xr_hbm, src_hbm, dst_hbm, att_hbm,
              accp_out,
              srcb, dstb, dstb2, xlrows, xrrows, cbuf2, attv, semx, semr,
              acc_sh):
    c = lax.axis_index("c")
    s = lax.axis_index("s")

    pltpu.sync_copy(att_hbm, attv)

    def zrow(i, carry):
        for q in range(8):
            cbuf2[i, pl.ds(16 * q, 16)] = jnp.zeros((16,), jnp.float32)
        return carry

    lax.fori_loop(0, CH, zrow, 0)
    for t in range(RPTB // CH):
        pltpu.sync_copy(cbuf2, acc_sh.at[pl.ds(s * RPTB + t * CH, CH)])
    rem = RPTB % CH
    if rem:
        off = RPTB - rem
        pltpu.sync_copy(cbuf2.at[pl.ds(0, rem)],
                        acc_sh.at[pl.ds(s * RPTB + off, rem)])
    dbase = NB + s * DRPT
    for t in range(DRPT // CH):
        pltpu.sync_copy(cbuf2, acc_sh.at[pl.ds(dbase + t * CH, CH)])
    drem = DRPT % CH
    if drem:
        pltpu.sync_copy(cbuf2.at[pl.ds(0, drem)],
                        acc_sh.at[pl.ds(dbase + DRPT - drem, drem)])
    plsc.subcore_barrier()

    attrows = [attv[pl.ds(h * DH, DH)] for h in range(HEADS)]
    lane = lax.iota(jnp.int32, 16)

    base0 = c * (NS * EPT) + s * EPT
    zv16 = jnp.zeros((16,), jnp.float32)

    def chunk(k, carry):
        base = base0 + k * CH
        pltpu.sync_copy(src_hbm.at[pl.ds(base, CH)], srcb)
        pltpu.sync_copy(dst_hbm.at[pl.ds(base, CH)], dstb)
        d1 = pltpu.async_copy(xl_hbm.at[srcb], xlrows, semx)
        d2 = pltpu.async_copy(xr_hbm.at[dstb], xrrows, semr)
        d1.wait()
        d2.wait()

        def group(g, carry2):
            dstv = dstb[pl.ds(g * 16, 16)]
            dstb2[pl.ds(g * 16, 16)] = NB + lax.shift_right_logical(dstv, 3)
            for l in range(16):
                j = g * 16 + l
                dj = dstv[l]
                bj = lax.rem(dj, 8)
                acc = jnp.zeros((16,), jnp.float32)
                lvs = []
                for h in range(HEADS):
                    sl = pl.ds(16 * h, 16)
                    lv = xlrows[j, sl]
                    rv = xrrows[j, sl]
                    m = lv + rv
                    lk = jnp.maximum(m, 0.2 * m)
                    eh = jnp.sum(lk * attrows[h])
                    acc = acc + jnp.where(lane == h, eh, 0.0)
                    lvs.append(lv)
                exrow = jnp.exp(acc)
                for h in range(HEADS):
                    xlrows[j, pl.ds(16 * h, 16)] = lvs[h] * exrow[h]
                for b in range(8):
                    cbuf2[j, pl.ds(16 * b, 16)] = jnp.where(bj == b, exrow, zv16)
            return carry2

        lax.fori_loop(0, CH // 16, group, 0)

        pltpu.sync_copy(xlrows, acc_sh.at[dstb], add=True)
        pltpu.sync_copy(cbuf2, acc_sh.at[dstb2], add=True)
        return carry

    lax.fori_loop(0, CHUNKS, chunk, 0)
    plsc.subcore_barrier()

    row0 = c * NBT + s * RPTB
    pltpu.sync_copy(acc_sh.at[pl.ds(s * RPTB, RPTB)], accp_out.at[pl.ds(row0, RPTB)])
    drow0 = c * NBT + NB + s * DRPT
    pltpu.sync_copy(acc_sh.at[pl.ds(NB + s * DRPT, DRPT)],
                    accp_out.at[pl.ds(drow0, DRPT)])


def _stage_b(xl, xr, src_pad, dst_pad, att1):
    f = pl.kernel(
        _sc1_body,
        out_type=jax.ShapeDtypeStruct((NC * NBT, HEADS * DH), jnp.float32),
        mesh=_mesh,
        compiler_params=_sc_params,
        scratch_types=[
            pltpu.VMEM((CH,), jnp.int32),
            pltpu.VMEM((CH,), jnp.int32),
            pltpu.VMEM((CH,), jnp.int32),
            pltpu.VMEM((CH, HEADS * DH), jnp.float32),
            pltpu.VMEM((CH, HEADS * DH), jnp.float32),
            pltpu.VMEM((CH, HEADS * DH), jnp.float32),
            pltpu.VMEM((HEADS * DH,), jnp.float32),
            pltpu.SemaphoreType.DMA,
            pltpu.SemaphoreType.DMA,
            pltpu.VMEM_SHARED((NBT, HEADS * DH), jnp.float32),
        ],
    )
    return f(xl, xr, src_pad, dst_pad, att1)


def _c_body(np_ref, dp_ref, b1_ref, w2_ref, hcat_ref):
    num = np_ref[0] + np_ref[1]
    den = (dp_ref[0] + dp_ref[1])[:, :HEADS]
    den16 = jnp.reshape(
        jnp.broadcast_to(den[:, :, None], (ROWBLK, HEADS, DH)),
        (ROWBLK, HEADS * DH))
    h = num / (den16 + 1e-16) + b1_ref[...]
    h = jnp.where(h > 0, h, jnp.exp(jnp.minimum(h, 0.0)) - 1.0)
    hcat_ref[...] = jnp.dot(h, w2_ref[...], preferred_element_type=jnp.float32)


def _stage_c(nump, denp, b1, W2cat):
    return pl.pallas_call(
        _c_body,
        grid=(NBLKS,),
        in_specs=[
            pl.BlockSpec((NC, ROWBLK, HEADS * DH), lambda i: (0, i, 0)),
            pl.BlockSpec((NC, ROWBLK, 16), lambda i: (0, i, 0)),
            pl.BlockSpec((1, HEADS * DH), lambda i: (0, 0)),
            pl.BlockSpec((HEADS * DH, 2), lambda i: (0, 0)),
        ],
        out_specs=pl.BlockSpec((ROWBLK, 2), lambda i: (i, 0)),
        out_shape=jax.ShapeDtypeStruct((NB, 2), jnp.float32),
    )(nump, denp, b1, W2cat)


def _sc2_body(hl_hbm, hr_hbm, src_hbm, dst_hbm, att2_hbm,
              num2_out, den2_out,
              hlv, hrv, srcb, dstb, exb2, cnb, attb, zv, sem1,
              num2_sh, den2_sh):
    c = lax.axis_index("c")
    s = lax.axis_index("s")

    pltpu.sync_copy(hl_hbm, hlv)
    pltpu.sync_copy(hr_hbm, hrv)
    pltpu.sync_copy(att2_hbm, attb)

    def zinit(i, carry):
        zv[pl.ds(i * 16, 16)] = jnp.zeros((16,), jnp.float32)
        return carry

    lax.fori_loop(0, RPTD // 16, zinit, 0)
    pltpu.sync_copy(zv, num2_sh.at[pl.ds(s * RPTD, RPTD)])
    pltpu.sync_copy(zv, den2_sh.at[pl.ds(s * RPTD, RPTD)])
    plsc.subcore_barrier()

    att2v = attb[...]
    base0 = c * (NS * EPT) + s * EPT

    def chunk(k, carry):
        base = base0 + k * CHD
        pltpu.sync_copy(src_hbm.at[pl.ds(base, CHD)], srcb)
        pltpu.sync_copy(dst_hbm.at[pl.ds(base, CHD)], dstb)

        def group(g, carry2):
            sl = pl.ds(g * 16, 16)
            sv = srcb[sl]
            dv = dstb[sl]
            a = plsc.load_gather(hlv, [sv])
            b = plsc.load_gather(hrv, [dv])
            m = a + b
            lk = jnp.maximum(m, 0.2 * m)
            ex = jnp.exp(lk * att2v)
            exb2[sl] = ex
            cnb[sl] = ex * a
            return carry2

        lax.fori_loop(0, CHD // 16, group, 0)

        pltpu.sync_copy(cnb, num2_sh.at[dstb], add=True)
        pltpu.sync_copy(exb2, den2_sh.at[dstb], add=True)
        return carry

    lax.fori_loop(0, CHUNKSD, chunk, 0)
    plsc.subcore_barrier()

    row0 = c * ND + s * RPTD
    pltpu.sync_copy(num2_sh.at[pl.ds(s * RPTD, RPTD)], num2_out.at[pl.ds(row0, RPTD)])
    pltpu.sync_copy(den2_sh.at[pl.ds(s * RPTD, RPTD)], den2_out.at[pl.ds(row0, RPTD)])


def _stage_d(hl, hr, src_pad, dst_pad, att2v):
    f = pl.kernel(
        _sc2_body,
        out_type=[
            jax.ShapeDtypeStruct((NC * ND,), jnp.float32),
            jax.ShapeDtypeStruct((NC * ND,), jnp.float32),
        ],
        mesh=_mesh,
        compiler_params=_sc_params,
        scratch_types=[
            pltpu.VMEM((ND,), jnp.float32),
            pltpu.VMEM((ND,), jnp.float32),
            pltpu.VMEM((CHD,), jnp.int32),
            pltpu.VMEM((CHD,), jnp.int32),
            pltpu.VMEM((CHD,), jnp.float32),
            pltpu.VMEM((CHD,), jnp.float32),
            pltpu.VMEM((16,), jnp.float32),
            pltpu.VMEM((RPTD,), jnp.float32),
            pltpu.SemaphoreType.DMA,
            pltpu.VMEM_SHARED((ND,), jnp.float32),
            pltpu.VMEM_SHARED((ND,), jnp.float32),
        ],
    )
    return f(hl, hr, src_pad, dst_pad, att2v)


def _e_body(n2_ref, d2_ref, b2_ref, batch_ref, o_ref):
    nv = jnp.sum(n2_ref[...], axis=0, keepdims=True)
    dv = jnp.sum(d2_ref[...], axis=0, keepdims=True)
    nodeval = nv / (dv + 1e-16) + b2_ref[...]
    gids = lax.broadcasted_iota(jnp.int32, (G, ND), 0)
    oh = (gids == batch_ref[...]).astype(jnp.float32)
    out = lax.dot_general(oh, nodeval, (((1,), (1,)), ((), ())),
                          preferred_element_type=jnp.float32)
    o_ref[...] = out


def _stage_e(num2p, den2p, b2, batch_pad):
    return pl.pallas_call(
        _e_body,
        out_shape=jax.ShapeDtypeStruct((G, 1), jnp.float32),
    )(num2p, den2p, b2, batch_pad)


def kernel(x, edge_index, batch, edge_attr, W1l, W1r, att1, b1, W2l, W2r, att2, b2):
    loop = jnp.arange(N, dtype=edge_index.dtype)
    padv = jnp.full((EPAD - (E + N),), N, dtype=edge_index.dtype)
    src_pad = jnp.concatenate([edge_index[0], loop, padv])
    dst_pad = jnp.concatenate([edge_index[1], loop, padv])

    x_pad = jnp.pad(x, ((0, NB - N), (0, 0)))
    att2v = jnp.full((16,), att2[0, 0], jnp.float32)
    W2cat = jnp.concatenate([W2l, W2r], axis=1)
    batch_pad = jnp.concatenate(
        [batch, jnp.full((ND - N,), G, batch.dtype)]).reshape(1, ND)

    xl, xr = _stage_a(x_pad, W1l, W1r)

    accp = _stage_b(xl, xr, src_pad, dst_pad, att1.reshape(HEADS * DH))
    accp = accp.reshape(NC, NBT, HEADS * DH)
    nump = accp[:, :NB, :]
    denp = accp[:, NB:, :].reshape(NC, DRB * 8, 16)[:, :NB, :]

    hcat = _stage_c(nump, denp, b1.reshape(1, HEADS * DH), W2cat)
    hl = jnp.pad(hcat[:, 0], (0, ND - NB))
    hr = jnp.pad(hcat[:, 1], (0, ND - NB))

    num2p, den2p = _stage_d(hl, hr, src_pad, dst_pad, att2v)

    out = _stage_e(num2p.reshape(NC, ND), den2p.reshape(NC, ND),
                   b2.reshape(1, 1), batch_pad)
    return out


# speedup vs baseline: 1.2437x; 1.0730x over previous
import jax
import jax.numpy as jnp
from jax import lax
from jax.experimental import pallas as pl
from jax.experimental.pallas import tpu as pltpu
import jax.experimental.pallas.tpu_sc as plsc

N = 10000
E = 320000
DIN = 128
DH = 16
HEADS = 8
G = 64

NC = 2
NS = 16
CH = 96
CHUNKS = 108
CHD = 128
CHUNKSD = 81
EPT = CH * CHUNKS
EPAD = EPT * NC * NS

NB = 10112
RPTB = NB // NS
ROWBLK = 128
NBLKS = NB // ROWBLK

DRB = 1280
DRPT = DRB // NS
NBT = NB + DRB

ND = 10240
RPTD = ND // NS

_mesh = plsc.VectorSubcoreMesh(core_axis_name="c", subcore_axis_name="s",
                               num_cores=NC, num_subcores=NS)
_sc_params = pltpu.CompilerParams(needs_layout_passes=False)


def _mm_body(x_ref, wl_ref, wr_ref, ol_ref, or_ref):
    xb = x_ref[...]
    ol_ref[...] = jnp.dot(xb, wl_ref[...], preferred_element_type=jnp.float32)
    or_ref[...] = jnp.dot(xb, wr_ref[...], preferred_element_type=jnp.float32)


def _stage_a(x_pad, W1l, W1r):
    return pl.pallas_call(
        _mm_body,
        grid=(NBLKS,),
        in_specs=[
            pl.BlockSpec((ROWBLK, DIN), lambda i: (i, 0)),
            pl.BlockSpec((DIN, HEADS * DH), lambda i: (0, 0)),
            pl.BlockSpec((DIN, HEADS * DH), lambda i: (0, 0)),
        ],
        out_specs=[
            pl.BlockSpec((ROWBLK, HEADS * DH), lambda i: (i, 0)),
            pl.BlockSpec((ROWBLK, HEADS * DH), lambda i: (i, 0)),
        ],
        out_shape=[
            jax.ShapeDtypeStruct((NB, HEADS * DH), jnp.float32),
            jax.ShapeDtypeStruct((NB, HEADS * DH), jnp.float32),
        ],
    )(x_pad, W1l, W1r)


def _sc1_body(xl_hbm, xr_hbm, src_hbm, dst_hbm, att_hbm,
              accp_out,
              srcb, dstb, dstb2, xlrows, xrrows, cbuf2, attv, semx, semr,
              acc_sh):
    c = lax.axis_index("c")
    s = lax.axis_index("s")

    pltpu.sync_copy(att_hbm, attv)

    def zrow(i, carry):
        for q in range(8):
            cbuf2[i, pl.ds(16 * q, 16)] = jnp.zeros((16,), jnp.float32)
        return carry

    lax.fori_loop(0, CH, zrow, 0)
    for t in range(RPTB // CH):
        pltpu.sync_copy(cbuf2, acc_sh.at[pl.ds(s * RPTB + t * CH, CH)])
    rem = RPTB % CH
    if rem:
        off = RPTB - rem
        pltpu.sync_copy(cbuf2.at[pl.ds(0, rem)],
                        acc_sh.at[pl.ds(s * RPTB + off, rem)])
    dbase = NB + s * DRPT
    for t in range(DRPT // CH):
        pltpu.sync_copy(cbuf2, acc_sh.at[pl.ds(dbase + t * CH, CH)])
    drem = DRPT % CH
    if drem:
        pltpu.sync_copy(cbuf2.at[pl.ds(0, drem)],
                        acc_sh.at[pl.ds(dbase + DRPT - drem, drem)])
    plsc.subcore_barrier()

    attrows = [attv[pl.ds(h * DH, DH)] for h in range(HEADS)]
    lane = lax.iota(jnp.int32, 16)

    base0 = c * (NS * EPT) + s * EPT
    zv16 = jnp.zeros((16,), jnp.float32)

    def chunk(k, carry):
        base = base0 + k * CH
        pltpu.sync_copy(src_hbm.at[pl.ds(base, CH)], srcb)
        pltpu.sync_copy(dst_hbm.at[pl.ds(base, CH)], dstb)
        d1 = pltpu.async_copy(xl_hbm.at[srcb], xlrows, semx)
        d2 = pltpu.async_copy(xr_hbm.at[dstb], xrrows, semr)
        d1.wait()
        d2.wait()

        def group(g, carry2):
            dstv = dstb[pl.ds(g * 16, 16)]
            dstb2[pl.ds(g * 16, 16)] = NB + lax.shift_right_logical(dstv, 3)
            for l in range(16):
                j = g * 16 + l
                dj = dstv[l]
                bj = lax.rem(dj, 8)
                acc = jnp.zeros((16,), jnp.float32)
                lvs = []
                for h in range(HEADS):
                    sl = pl.ds(16 * h, 16)
                    lv = xlrows[j, sl]
                    rv = xrrows[j, sl]
                    m = lv + rv
                    lk = jnp.maximum(m, 0.2 * m)
                    eh = jnp.sum(lk * attrows[h])
                    acc = acc + jnp.where(lane == h, eh, 0.0)
                    lvs.append(lv)
                exrow = jnp.exp(acc)
                for h in range(HEADS):
                    xlrows[j, pl.ds(16 * h, 16)] = lvs[h] * exrow[h]
                for b in range(8):
                    cbuf2[j, pl.ds(16 * b, 16)] = jnp.where(bj == b, exrow, zv16)
            return carry2

        lax.fori_loop(0, CH // 16, group, 0)

        pltpu.sync_copy(xlrows, acc_sh.at[dstb], add=True)
        pltpu.sync_copy(cbuf2, acc_sh.at[dstb2], add=True)
        return carry

    lax.fori_loop(0, CHUNKS, chunk, 0)
    plsc.subcore_barrier()

    row0 = c * NBT + s * RPTB
    pltpu.sync_copy(acc_sh.at[pl.ds(s * RPTB, RPTB)], accp_out.at[pl.ds(row0, RPTB)])
    drow0 = c * NBT + NB + s * DRPT
    pltpu.sync_copy(acc_sh.at[pl.ds(NB + s * DRPT, DRPT)],
                    accp_out.at[pl.ds(drow0, DRPT)])


def _stage_b(xl, xr, src_pad, dst_pad, att1):
    f = pl.kernel(
        _sc1_body,
        out_type=jax.ShapeDtypeStruct((NC * NBT, HEADS * DH), jnp.float32),
        mesh=_mesh,
        compiler_params=_sc_params,
        scratch_types=[
            pltpu.VMEM((CH,), jnp.int32),
            pltpu.VMEM((CH,), jnp.int32),
            pltpu.VMEM((CH,), jnp.int32),
            pltpu.VMEM((CH, HEADS * DH), jnp.float32),
            pltpu.VMEM((CH, HEADS * DH), jnp.float32),
            pltpu.VMEM((CH, HEADS * DH), jnp.float32),
            pltpu.VMEM((HEADS * DH,), jnp.float32),
            pltpu.SemaphoreType.DMA,
            pltpu.SemaphoreType.DMA,
            pltpu.VMEM_SHARED((NBT, HEADS * DH), jnp.float32),
        ],
    )
    return f(xl, xr, src_pad, dst_pad, att1)


def _c_body(np_ref, dp_ref, b1_ref, w2_ref, hcat_ref):
    num = np_ref[0] + np_ref[1]
    den = (dp_ref[0] + dp_ref[1])[:, :HEADS]
    den16 = jnp.reshape(
        jnp.broadcast_to(den[:, :, None], (ROWBLK, HEADS, DH)),
        (ROWBLK, HEADS * DH))
    h = num / (den16 + 1e-16) + b1_ref[...]
    h = jnp.where(h > 0, h, jnp.exp(jnp.minimum(h, 0.0)) - 1.0)
    hcat_ref[...] = jnp.dot(h, w2_ref[...], preferred_element_type=jnp.float32)


def _stage_c(nump, denp, b1, W2cat):
    return pl.pallas_call(
        _c_body,
        grid=(NBLKS,),
        in_specs=[
            pl.BlockSpec((NC, ROWBLK, HEADS * DH), lambda i: (0, i, 0)),
            pl.BlockSpec((NC, ROWBLK, 16), lambda i: (0, i, 0)),
            pl.BlockSpec((1, HEADS * DH), lambda i: (0, 0)),
            pl.BlockSpec((HEADS * DH, 2), lambda i: (0, 0)),
        ],
        out_specs=pl.BlockSpec((ROWBLK, 2), lambda i: (i, 0)),
        out_shape=jax.ShapeDtypeStruct((NB, 2), jnp.float32),
    )(nump, denp, b1, W2cat)


def _sc2_body(hl_hbm, hr_hbm, src_hbm, dst_hbm, att2_hbm,
              num2_out, den2_out,
              hlv, hrv, srcb, dstb, exb2, cnb, attb, zv, sem1,
              num2_sh, den2_sh):
    c = lax.axis_index("c")
    s = lax.axis_index("s")

    pltpu.sync_copy(hl_hbm, hlv)
    pltpu.sync_copy(hr_hbm, hrv)
    pltpu.sync_copy(att2_hbm, attb)

    def zinit(i, carry):
        zv[pl.ds(i * 16, 16)] = jnp.zeros((16,), jnp.float32)
        return carry

    lax.fori_loop(0, RPTD // 16, zinit, 0)
    pltpu.sync_copy(zv, num2_sh.at[pl.ds(s * RPTD, RPTD)])
    pltpu.sync_copy(zv, den2_sh.at[pl.ds(s * RPTD, RPTD)])
    plsc.subcore_barrier()

    att2v = attb[...]
    base0 = c * (NS * EPT) + s * EPT

    def chunk(k, carry):
        base = base0 + k * CHD
        pltpu.sync_copy(src_hbm.at[pl.ds(base, CHD)], srcb)
        pltpu.sync_copy(dst_hbm.at[pl.ds(base, CHD)], dstb)

        def group(g, carry2):
            sl = pl.ds(g * 16, 16)
            sv = srcb[sl]
            dv = dstb[sl]
            a = plsc.load_gather(hlv, [sv])
            b = plsc.load_gather(hrv, [dv])
            m = a + b
            lk = jnp.maximum(m, 0.2 * m)
            ex = jnp.exp(lk * att2v)
            exb2[sl] = ex
            cnb[sl] = ex * a
            return carry2

        lax.fori_loop(0, CHD // 16, group, 0)

        pltpu.sync_copy(cnb, num2_sh.at[dstb], add=True)
        pltpu.sync_copy(exb2, den2_sh.at[dstb], add=True)
        return carry

    lax.fori_loop(0, CHUNKSD, chunk, 0)
    plsc.subcore_barrier()

    row0 = c * ND + s * RPTD
    pltpu.sync_copy(num2_sh.at[pl.ds(s * RPTD, RPTD)], num2_out.at[pl.ds(row0, RPTD)])
    pltpu.sync_copy(den2_sh.at[pl.ds(s * RPTD, RPTD)], den2_out.at[pl.ds(row0, RPTD)])


def _stage_d(hl, hr, src_pad, dst_pad, att2v):
    f = pl.kernel(
        _sc2_body,
        out_type=[
            jax.ShapeDtypeStruct((NC * ND,), jnp.float32),
            jax.ShapeDtypeStruct((NC * ND,), jnp.float32),
        ],
        mesh=_mesh,
        compiler_params=_sc_params,
        scratch_types=[
            pltpu.VMEM((ND,), jnp.float32),
            pltpu.VMEM((ND,), jnp.float32),
            pltpu.VMEM((CHD,), jnp.int32),
            pltpu.VMEM((CHD,), jnp.int32),
            pltpu.VMEM((CHD,), jnp.float32),
            pltpu.VMEM((CHD,), jnp.float32),
            pltpu.VMEM((16,), jnp.float32),
            pltpu.VMEM((RPTD,), jnp.float32),
            pltpu.SemaphoreType.DMA,
            pltpu.VMEM_SHARED((ND,), jnp.float32),
            pltpu.VMEM_SHARED((ND,), jnp.float32),
        ],
    )
    return f(hl, hr, src_pad, dst_pad, att2v)


def _e_body(n2_ref, d2_ref, b2_ref, batch_ref, o_ref):
    nv = jnp.sum(n2_ref[...], axis=0, keepdims=True)
    dv = jnp.sum(d2_ref[...], axis=0, keepdims=True)
    nodeval = nv / (dv + 1e-16) + b2_ref[...]
    gids = lax.broadcasted_iota(jnp.int32, (G, ND), 0)
    oh = (gids == batch_ref[...]).astype(jnp.float32)
    out = lax.dot_general(oh, nodeval, (((1,), (1,)), ((), ())),
                          preferred_element_type=jnp.float32)
    o_ref[...] = out


def _stage_e(num2p, den2p, b2, batch_pad):
    return pl.pallas_call(
        _e_body,
        out_shape=jax.ShapeDtypeStruct((G, 1), jnp.float32),
    )(num2p, den2p, b2, batch_pad)


def kernel(x, edge_index, batch, edge_attr, W1l, W1r, att1, b1, W2l, W2r, att2, b2):
    loop = jnp.arange(N, dtype=edge_index.dtype)
    padv = jnp.full((EPAD - (E + N),), N, dtype=edge_index.dtype)
    src_pad = jnp.concatenate([edge_index[0], loop, padv])
    dst_pad = jnp.concatenate([edge_index[1], loop, padv])

    x_pad = jnp.pad(x, ((0, NB - N), (0, 0)))
    att2v = jnp.full((16,), att2[0, 0], jnp.float32)
    W2cat = jnp.concatenate([W2l, W2r], axis=1)
    batch_pad = jnp.concatenate(
        [batch, jnp.full((ND - N,), G, batch.dtype)]).reshape(1, ND)

    xl, xr = _stage_a(x_pad, W1l, W1r)

    accp = _stage_b(xl, xr, src_pad, dst_pad, att1.reshape(HEADS * DH))
    accp = accp.reshape(NC, NBT, HEADS * DH)
    nump = accp[:, :NB, :]
    denp = accp[:, NB:, :].reshape(NC, DRB * 8, 16)[:, :NB, :]

    hcat = _stage_c(nump, denp, b1.reshape(1, HEADS * DH), W2cat)
    hl = jnp.pad(hcat[:, 0], (0, ND - NB))
    hr = jnp.pad(hcat[:, 1], (0, ND - NB))

    num2p, den2p = _stage_d(hl, hr, src_pad, dst_pad, att2v)

    out = _stage_e(num2p.reshape(NC, ND), den2p.reshape(NC, ND),
                   b2.reshape(1, 1), batch_pad)
    return out


# speedup vs baseline: 1.2836x; 1.0321x over previous
import jax
import jax.numpy as jnp
from jax import lax
from jax.experimental import pallas as pl
from jax.experimental.pallas import tpu as pltpu
import jax.experimental.pallas.tpu_sc as plsc

N = 10000
E = 320000
DIN = 128
DH = 16
HEADS = 8
G = 64

NC = 2
NS = 16
CH = 96
CHUNKS = 108
CHD = 192
CHUNKSD = 54
EPT = CH * CHUNKS
EPAD = EPT * NC * NS

NB = 10112
RPTB = NB // NS
ROWBLK = 128
NBLKS = NB // ROWBLK

DRB = 1280
DRPT = DRB // NS
NBT = NB + DRB

ND = 10240
RPTD = ND // NS

_mesh = plsc.VectorSubcoreMesh(core_axis_name="c", subcore_axis_name="s",
                               num_cores=NC, num_subcores=NS)
_sc_params = pltpu.CompilerParams(needs_layout_passes=False)


def _mm_body(x_ref, wl_ref, wr_ref, ol_ref, or_ref):
    xb = x_ref[...]
    ol_ref[...] = jnp.dot(xb, wl_ref[...], preferred_element_type=jnp.float32)
    or_ref[...] = jnp.dot(xb, wr_ref[...], preferred_element_type=jnp.float32)


def _stage_a(x_pad, W1l, W1r):
    return pl.pallas_call(
        _mm_body,
        grid=(NBLKS,),
        in_specs=[
            pl.BlockSpec((ROWBLK, DIN), lambda i: (i, 0)),
            pl.BlockSpec((DIN, HEADS * DH), lambda i: (0, 0)),
            pl.BlockSpec((DIN, HEADS * DH), lambda i: (0, 0)),
        ],
        out_specs=[
            pl.BlockSpec((ROWBLK, HEADS * DH), lambda i: (i, 0)),
            pl.BlockSpec((ROWBLK, HEADS * DH), lambda i: (i, 0)),
        ],
        out_shape=[
            jax.ShapeDtypeStruct((NB, HEADS * DH), jnp.float32),
            jax.ShapeDtypeStruct((NB, HEADS * DH), jnp.float32),
        ],
    )(x_pad, W1l, W1r)


def _sc1_body(xl_hbm, xr_hbm, src_hbm, dst_hbm, att_hbm,
              accp_out,
              srcb, dstb, dstb2, xlrows, xrrows, cbuf2, attv, semx, semr,
              acc_sh):
    c = lax.axis_index("c")
    s = lax.axis_index("s")

    pltpu.sync_copy(att_hbm, attv)

    def zrow(i, carry):
        for q in range(8):
            cbuf2[i, pl.ds(16 * q, 16)] = jnp.zeros((16,), jnp.float32)
        return carry

    lax.fori_loop(0, CH, zrow, 0)
    for t in range(RPTB // CH):
        pltpu.sync_copy(cbuf2, acc_sh.at[pl.ds(s * RPTB + t * CH, CH)])
    rem = RPTB % CH
    if rem:
        off = RPTB - rem
        pltpu.sync_copy(cbuf2.at[pl.ds(0, rem)],
                        acc_sh.at[pl.ds(s * RPTB + off, rem)])
    dbase = NB + s * DRPT
    for t in range(DRPT // CH):
        pltpu.sync_copy(cbuf2, acc_sh.at[pl.ds(dbase + t * CH, CH)])
    drem = DRPT % CH
    if drem:
        pltpu.sync_copy(cbuf2.at[pl.ds(0, drem)],
                        acc_sh.at[pl.ds(dbase + DRPT - drem, drem)])
    plsc.subcore_barrier()

    attrows = [attv[pl.ds(h * DH, DH)] for h in range(HEADS)]
    lane = lax.iota(jnp.int32, 16)

    base0 = c * (NS * EPT) + s * EPT
    zv16 = jnp.zeros((16,), jnp.float32)

    def chunk(k, carry):
        base = base0 + k * CH
        pltpu.sync_copy(src_hbm.at[pl.ds(base, CH)], srcb)
        pltpu.sync_copy(dst_hbm.at[pl.ds(base, CH)], dstb)
        d1 = pltpu.async_copy(xl_hbm.at[srcb], xlrows, semx)
        d2 = pltpu.async_copy(xr_hbm.at[dstb], xrrows, semr)
        d1.wait()
        d2.wait()

        def group(g, carry2):
            dstv = dstb[pl.ds(g * 16, 16)]
            dstb2[pl.ds(g * 16, 16)] = NB + lax.shift_right_logical(dstv, 3)
            for l in range(16):
                j = g * 16 + l
                dj = dstv[l]
                bj = lax.rem(dj, 8)
                acc = jnp.zeros((16,), jnp.float32)
                lvs = []
                for h in range(HEADS):
                    sl = pl.ds(16 * h, 16)
                    lv = xlrows[j, sl]
                    rv = xrrows[j, sl]
                    m = lv + rv
                    lk = jnp.maximum(m, 0.2 * m)
                    eh = jnp.sum(lk * attrows[h])
                    acc = acc + jnp.where(lane == h, eh, 0.0)
                    lvs.append(lv)
                exrow = jnp.exp(acc)
                for h in range(HEADS):
                    xlrows[j, pl.ds(16 * h, 16)] = lvs[h] * exrow[h]
                for b in range(8):
                    cbuf2[j, pl.ds(16 * b, 16)] = jnp.where(bj == b, exrow, zv16)
            return carry2

        lax.fori_loop(0, CH // 16, group, 0)

        pltpu.sync_copy(xlrows, acc_sh.at[dstb], add=True)
        pltpu.sync_copy(cbuf2, acc_sh.at[dstb2], add=True)
        return carry

    lax.fori_loop(0, CHUNKS, chunk, 0)
    plsc.subcore_barrier()

    row0 = c * NBT + s * RPTB
    pltpu.sync_copy(acc_sh.at[pl.ds(s * RPTB, RPTB)], accp_out.at[pl.ds(row0, RPTB)])
    drow0 = c * NBT + NB + s * DRPT
    pltpu.sync_copy(acc_sh.at[pl.ds(NB + s * DRPT, DRPT)],
                    accp_out.at[pl.ds(drow0, DRPT)])


def _stage_b(xl, xr, src_pad, dst_pad, att1):
    f = pl.kernel(
        _sc1_body,
        out_type=jax.ShapeDtypeStruct((NC * NBT, HEADS * DH), jnp.float32),
        mesh=_mesh,
        compiler_params=_sc_params,
        scratch_types=[
            pltpu.VMEM((CH,), jnp.int32),
            pltpu.VMEM((CH,), jnp.int32),
            pltpu.VMEM((CH,), jnp.int32),
            pltpu.VMEM((CH, HEADS * DH), jnp.float32),
            pltpu.VMEM((CH, HEADS * DH), jnp.float32),
            pltpu.VMEM((CH, HEADS * DH), jnp.float32),
            pltpu.VMEM((HEADS * DH,), jnp.float32),
            pltpu.SemaphoreType.DMA,
            pltpu.SemaphoreType.DMA,
            pltpu.VMEM_SHARED((NBT, HEADS * DH), jnp.float32),
        ],
    )
    return f(xl, xr, src_pad, dst_pad, att1)


def _c_body(np_ref, dp_ref, b1_ref, w2_ref, hcat_ref):
    num = np_ref[0] + np_ref[1]
    den = (dp_ref[0] + dp_ref[1])[:, :HEADS]
    den16 = jnp.reshape(
        jnp.broadcast_to(den[:, :, None], (ROWBLK, HEADS, DH)),
        (ROWBLK, HEADS * DH))
    h = num / (den16 + 1e-16) + b1_ref[...]
    h = jnp.where(h > 0, h, jnp.exp(jnp.minimum(h, 0.0)) - 1.0)
    hcat_ref[...] = jnp.dot(h, w2_ref[...], preferred_element_type=jnp.float32)


def _stage_c(nump, denp, b1, W2cat):
    return pl.pallas_call(
        _c_body,
        grid=(NBLKS,),
        in_specs=[
            pl.BlockSpec((NC, ROWBLK, HEADS * DH), lambda i: (0, i, 0)),
            pl.BlockSpec((NC, ROWBLK, 16), lambda i: (0, i, 0)),
            pl.BlockSpec((1, HEADS * DH), lambda i: (0, 0)),
            pl.BlockSpec((HEADS * DH, 2), lambda i: (0, 0)),
        ],
        out_specs=pl.BlockSpec((ROWBLK, 2), lambda i: (i, 0)),
        out_shape=jax.ShapeDtypeStruct((NB, 2), jnp.float32),
    )(nump, denp, b1, W2cat)


def _sc2_body(hl_hbm, hr_hbm, src_hbm, dst_hbm, att2_hbm,
              num2_out, den2_out,
              hlv, hrv, srcb, dstb, exb2, cnb, attb, zv, sem1,
              num2_sh, den2_sh):
    c = lax.axis_index("c")
    s = lax.axis_index("s")

    pltpu.sync_copy(hl_hbm, hlv)
    pltpu.sync_copy(hr_hbm, hrv)
    pltpu.sync_copy(att2_hbm, attb)

    def zinit(i, carry):
        zv[pl.ds(i * 16, 16)] = jnp.zeros((16,), jnp.float32)
        return carry

    lax.fori_loop(0, RPTD // 16, zinit, 0)
    pltpu.sync_copy(zv, num2_sh.at[pl.ds(s * RPTD, RPTD)])
    pltpu.sync_copy(zv, den2_sh.at[pl.ds(s * RPTD, RPTD)])
    plsc.subcore_barrier()

    att2v = attb[...]
    base0 = c * (NS * EPT) + s * EPT

    def chunk(k, carry):
        base = base0 + k * CHD
        pltpu.sync_copy(src_hbm.at[pl.ds(base, CHD)], srcb)
        pltpu.sync_copy(dst_hbm.at[pl.ds(base, CHD)], dstb)

        def group(g, carry2):
            sl = pl.ds(g * 16, 16)
            sv = srcb[sl]
            dv = dstb[sl]
            a = plsc.load_gather(hlv, [sv])
            b = plsc.load_gather(hrv, [dv])
            m = a + b
            lk = jnp.maximum(m, 0.2 * m)
            ex = jnp.exp(lk * att2v)
            exb2[sl] = ex
            cnb[sl] = ex * a
            return carry2

        lax.fori_loop(0, CHD // 16, group, 0)

        pltpu.sync_copy(cnb, num2_sh.at[dstb], add=True)
        pltpu.sync_copy(exb2, den2_sh.at[dstb], add=True)
        return carry

    lax.fori_loop(0, CHUNKSD, chunk, 0)
    plsc.subcore_barrier()

    row0 = c * ND + s * RPTD
    pltpu.sync_copy(num2_sh.at[pl.ds(s * RPTD, RPTD)], num2_out.at[pl.ds(row0, RPTD)])
    pltpu.sync_copy(den2_sh.at[pl.ds(s * RPTD, RPTD)], den2_out.at[pl.ds(row0, RPTD)])


def _stage_d(hl, hr, src_pad, dst_pad, att2v):
    f = pl.kernel(
        _sc2_body,
        out_type=[
            jax.ShapeDtypeStruct((NC * ND,), jnp.float32),
            jax.ShapeDtypeStruct((NC * ND,), jnp.float32),
        ],
        mesh=_mesh,
        compiler_params=_sc_params,
        scratch_types=[
            pltpu.VMEM((ND,), jnp.float32),
            pltpu.VMEM((ND,), jnp.float32),
            pltpu.VMEM((CHD,), jnp.int32),
            pltpu.VMEM((CHD,), jnp.int32),
            pltpu.VMEM((CHD,), jnp.float32),
            pltpu.VMEM((CHD,), jnp.float32),
            pltpu.VMEM((16,), jnp.float32),
            pltpu.VMEM((RPTD,), jnp.float32),
            pltpu.SemaphoreType.DMA,
            pltpu.VMEM_SHARED((ND,), jnp.float32),
            pltpu.VMEM_SHARED((ND,), jnp.float32),
        ],
    )
    return f(hl, hr, src_pad, dst_pad, att2v)


def _e_body(n2_ref, d2_ref, b2_ref, batch_ref, o_ref):
    nv = jnp.sum(n2_ref[...], axis=0, keepdims=True)
    dv = jnp.sum(d2_ref[...], axis=0, keepdims=True)
    nodeval = nv / (dv + 1e-16) + b2_ref[...]
    gids = lax.broadcasted_iota(jnp.int32, (G, ND), 0)
    oh = (gids == batch_ref[...]).astype(jnp.float32)
    out = lax.dot_general(oh, nodeval, (((1,), (1,)), ((), ())),
                          preferred_element_type=jnp.float32)
    o_ref[...] = out


def _stage_e(num2p, den2p, b2, batch_pad):
    return pl.pallas_call(
        _e_body,
        out_shape=jax.ShapeDtypeStruct((G, 1), jnp.float32),
    )(num2p, den2p, b2, batch_pad)


def kernel(x, edge_index, batch, edge_attr, W1l, W1r, att1, b1, W2l, W2r, att2, b2):
    loop = jnp.arange(N, dtype=edge_index.dtype)
    padv = jnp.full((EPAD - (E + N),), N, dtype=edge_index.dtype)
    src_pad = jnp.concatenate([edge_index[0], loop, padv])
    dst_pad = jnp.concatenate([edge_index[1], loop, padv])

    x_pad = jnp.pad(x, ((0, NB - N), (0, 0)))
    att2v = jnp.full((16,), att2[0, 0], jnp.float32)
    W2cat = jnp.concatenate([W2l, W2r], axis=1)
    batch_pad = jnp.concatenate(
        [batch, jnp.full((ND - N,), G, batch.dtype)]).reshape(1, ND)

    xl, xr = _stage_a(x_pad, W1l, W1r)

    accp = _stage_b(xl, xr, src_pad, dst_pad, att1.reshape(HEADS * DH))
    accp = accp.reshape(NC, NBT, HEADS * DH)
    nump = accp[:, :NB, :]
    denp = accp[:, NB:, :].reshape(NC, DRB * 8, 16)[:, :NB, :]

    hcat = _stage_c(nump, denp, b1.reshape(1, HEADS * DH), W2cat)
    hl = jnp.pad(hcat[:, 0], (0, ND - NB))
    hr = jnp.pad(hcat[:, 1], (0, ND - NB))

    num2p, den2p = _stage_d(hl, hr, src_pad, dst_pad, att2v)

    out = _stage_e(num2p.reshape(NC, ND), den2p.reshape(NC, ND),
                   b2.reshape(1, 1), batch_pad)
    return out
